# baseline (speedup 1.0000x reference)
import functools

import jax
import jax.numpy as jnp
from jax import lax
from jax.experimental import pallas as pl
from jax.experimental.pallas import tpu as pltpu
from jax.experimental.pallas import tpu_sc as plsc

_NUM_CORES = 2
_NUM_SUBCORES = 16
_NW = _NUM_CORES * _NUM_SUBCORES

_CHUNK = 128
_EMBED = 32


def _emb_kernel(x_hbm, table_hbm, out_hbm, idx_v, rows_v, sem):
  chunks_per_w = x_hbm.shape[1]
  wid = lax.axis_index("s") * _NUM_CORES + lax.axis_index("c")
  row0 = wid * chunks_per_w
  pltpu.sync_copy(x_hbm.at[wid], idx_v)

  @pl.loop(0, chunks_per_w)
  def _chunk(c):
    pltpu.async_copy(table_hbm.at[idx_v.at[c]], rows_v, sem).wait()
    pltpu.sync_copy(rows_v, out_hbm.at[pl.ds((row0 + c) * _CHUNK, _CHUNK)])


@jax.jit
def _run(x_flat, table):
  n_chunks_total = x_flat.shape[0] // _CHUNK
  chunks_per_w = n_chunks_total // _NW
  x2 = x_flat.reshape(_NW, chunks_per_w, _CHUNK)
  mesh = plsc.VectorSubcoreMesh(core_axis_name="c", subcore_axis_name="s")
  out = pl.kernel(
      _emb_kernel,
      out_type=jax.ShapeDtypeStruct((x_flat.shape[0], _EMBED), jnp.float32),
      mesh=mesh,
      compiler_params=pltpu.CompilerParams(use_tc_tiling_on_sc=False),
      scratch_types=[
          pltpu.VMEM((chunks_per_w, _CHUNK), jnp.int32),
          pltpu.VMEM((_CHUNK, _EMBED), jnp.float32),
          pltpu.SemaphoreType.DMA,
      ],
  )(x2, table)
  return out


def kernel(x, table):
  b, h = x.shape
  out = _run(x.reshape(b * h).astype(jnp.int32), table)
  return out.reshape(b, h, table.shape[1])


# speedup vs baseline: 1.0459x; 1.0459x over previous
import jax
import jax.numpy as jnp
from jax import lax
from jax.experimental import pallas as pl
from jax.experimental.pallas import tpu as pltpu
from jax.experimental.pallas import tpu_sc as plsc

_NUM_CORES = 2
_NUM_SUBCORES = 16
_NW = _NUM_CORES * _NUM_SUBCORES

_CHUNK = 800
_NCHUNK = 8
_EMBED = 32


def _emb_kernel(x_hbm, table_hbm, out_hbm, idx_v, rows_v, gs0, gs1, os0, os1):
  wid = lax.axis_index("s") * _NUM_CORES + lax.axis_index("c")
  row0 = wid * (_NCHUNK * _CHUNK)
  gsem = (gs0, gs1)
  osem = (os0, os1)
  pltpu.sync_copy(x_hbm.at[wid], idx_v)

  g_descs = [None] * _NCHUNK
  o_descs = [None] * _NCHUNK
  for c in range(_NCHUNK):
    b = c % 2
    if c >= 2:
      o_descs[c - 2].wait()
    g_descs[c] = pltpu.async_copy(
        table_hbm.at[idx_v.at[c]], rows_v.at[b], gsem[b])
    if c >= 1:
      g_descs[c - 1].wait()
      o_descs[c - 1] = pltpu.async_copy(
          rows_v.at[1 - b],
          out_hbm.at[pl.ds(row0 + (c - 1) * _CHUNK, _CHUNK)],
          osem[1 - b])
  last = _NCHUNK - 1
  g_descs[last].wait()
  o_descs[last] = pltpu.async_copy(
      rows_v.at[last % 2],
      out_hbm.at[pl.ds(row0 + last * _CHUNK, _CHUNK)],
      osem[last % 2])
  o_descs[last - 1].wait()
  o_descs[last].wait()


@jax.jit
def _run(x_flat, table):
  x3 = x_flat.reshape(_NW, _NCHUNK, _CHUNK)
  mesh = plsc.VectorSubcoreMesh(core_axis_name="c", subcore_axis_name="s")
  out = pl.kernel(
      _emb_kernel,
      out_type=jax.ShapeDtypeStruct((x_flat.shape[0], _EMBED), jnp.float32),
      mesh=mesh,
      compiler_params=pltpu.CompilerParams(use_tc_tiling_on_sc=False),
      scratch_types=[
          pltpu.VMEM((_NCHUNK, _CHUNK), jnp.int32),
          pltpu.VMEM((2, _CHUNK, _EMBED), jnp.float32),
          pltpu.SemaphoreType.DMA,
          pltpu.SemaphoreType.DMA,
          pltpu.SemaphoreType.DMA,
          pltpu.SemaphoreType.DMA,
      ],
  )(x3, table)
  return out


def kernel(x, table):
  b, h = x.shape
  out = _run(x.reshape(b * h).astype(jnp.int32), table)
  return out.reshape(b, h, table.shape[1])
